# Initial kernel scaffold; baseline (speedup 1.0000x reference)
import jax
import jax.numpy as jnp
from jax.experimental import pallas as pl


def kernel(inputs, adj, params):
    raise NotImplementedError("write your pallas kernel here")


# speedup vs baseline: 2.9666x; 2.9666x over previous
import jax
import jax.numpy as jnp
from jax.experimental import pallas as pl
from jax.experimental.pallas import tpu as pltpu

_N = 207
_NP = 208
_B = 8
_R = _B * _NP
_U = 64
_C = 128
_SEQ = 12
_HOR = 12
_M = 5


def _mm(a, b):
    return jnp.dot(a, b, preferred_element_type=jnp.float32)


def _fism_body(inp_ref, adj_ref,
               eWg0, ebg0, eWc0, ebc0, eWg1, ebg1, eWc1, ebc1,
               dWg0, dbg0, dWc0, dbc0, dWg1, dbg1, dWc1, dbc1,
               wproj, bproj,
               out_ref, h0_ref, h1_ref, src_ref, d1_ref, d2_ref):
    a = adj_ref[...]
    d1 = jnp.maximum(jnp.sum(a, axis=1, keepdims=True), 1e-8)
    sa = (a / d1).T
    at = a.T
    d2 = jnp.maximum(jnp.sum(at, axis=1, keepdims=True), 1e-8)
    sb = (at / d2).T

    def dconv(x0, w_ref, b_ref):
        src_ref[...] = x0
        acc = _mm(x0, w_ref[0])
        mi = 1
        for s in (sa, sb):
            def bstep(b, _):
                sl = pl.ds(b * _NP, _NP)
                x1b = _mm(s, src_ref[sl, :])
                d1_ref[sl, :] = x1b
                d2_ref[sl, :] = 2.0 * _mm(s, x1b) - src_ref[sl, :]
                return 0
            jax.lax.fori_loop(0, _B, bstep, 0, unroll=False)
            acc = acc + _mm(d1_ref[...], w_ref[mi]) + _mm(d2_ref[...], w_ref[mi + 1])
            mi += 2
        return acc + b_ref[...]

    zpad0 = jnp.zeros((_R, _C - 1 - _U), jnp.float32)

    def cell0(x, h, wg, bg, wc, bc):
        cat = jnp.concatenate([x, h, zpad0], axis=1)
        val = jax.nn.sigmoid(dconv(cat, wg, bg))
        r = val[:, :_U]
        u = val[:, _U:]
        cat2 = jnp.concatenate([x, r * h, zpad0], axis=1)
        c = jnp.tanh(dconv(cat2, wc, bc))
        return u * h + (1.0 - u) * c

    def cell1(x, h, wg, bg, wc, bc):
        cat = jnp.concatenate([x, h], axis=1)
        val = jax.nn.sigmoid(dconv(cat, wg, bg))
        r = val[:, :_U]
        u = val[:, _U:]
        cat2 = jnp.concatenate([x, r * h], axis=1)
        c = jnp.tanh(dconv(cat2, wc, bc))
        return u * h + (1.0 - u) * c

    h0_ref[...] = jnp.zeros((_R, _U), jnp.float32)
    h1_ref[...] = jnp.zeros((_R, _U), jnp.float32)

    def enc_body(t, carry):
        x = inp_ref[t]
        nh0 = cell0(x, h0_ref[...], eWg0, ebg0, eWc0, ebc0)
        nh1 = cell1(nh0, h1_ref[...], eWg1, ebg1, eWc1, ebc1)
        h0_ref[...] = nh0
        h1_ref[...] = nh1
        return carry

    jax.lax.fori_loop(0, _SEQ, enc_body, 0, unroll=False)

    def dec_body(t, dec_in):
        nh0 = cell0(dec_in, h0_ref[...], dWg0, dbg0, dWc0, dbc0)
        nh1 = cell1(nh0, h1_ref[...], dWg1, dbg1, dWc1, dbc1)
        h0_ref[...] = nh0
        h1_ref[...] = nh1
        proj = _mm(nh1, wproj[...]) + bproj[...]
        out_ref[t] = proj
        return proj

    jax.lax.fori_loop(0, _HOR, dec_body,
                      jnp.zeros((_R, 1), jnp.float32), unroll=False)


def _reorder_w(w, cin, out):
    w = w.reshape(cin, _M, out).transpose(1, 0, 2)
    return jnp.pad(w, ((0, 0), (0, _C - cin), (0, 0)))


def kernel(inputs, adj, params):
    inp = jnp.pad(inputs, ((0, 0), (0, 0), (0, _NP - _N)))
    inp = inp.reshape(_SEQ, _R, 1)
    adj_p = jnp.pad(adj, ((0, _NP - _N), (0, _NP - _N)))

    args = [inp, adj_p]
    for mdl in ("enc", "dec"):
        for l in range(2):
            cin = (1 if l == 0 else _U) + _U
            args.append(_reorder_w(params[f"{mdl}_Wg{l}"], cin, 2 * _U))
            args.append(params[f"{mdl}_bg{l}"].reshape(1, 2 * _U))
            args.append(_reorder_w(params[f"{mdl}_Wc{l}"], cin, _U))
            args.append(params[f"{mdl}_bc{l}"].reshape(1, _U))
    args.append(params["W_proj"])
    args.append(params["b_proj"].reshape(1, 1))

    out = pl.pallas_call(
        _fism_body,
        out_shape=jax.ShapeDtypeStruct((_HOR, _R, 1), jnp.float32),
        scratch_shapes=[pltpu.VMEM((_R, _U), jnp.float32),
                        pltpu.VMEM((_R, _U), jnp.float32),
                        pltpu.VMEM((_R, _C), jnp.float32),
                        pltpu.VMEM((_R, _C), jnp.float32),
                        pltpu.VMEM((_R, _C), jnp.float32)],
    )(*args)

    return out.reshape(_HOR, _B, _NP)[:, :, :_N]


# speedup vs baseline: 5.7819x; 1.9490x over previous
import jax
import jax.numpy as jnp
from jax.experimental import pallas as pl
from jax.experimental.pallas import tpu as pltpu

_N = 207
_NP = 208
_B = 8
_R = _B * _NP
_U = 64
_C = 128
_SEQ = 12
_HOR = 12
_M = 5


def _mm(a, b):
    return jnp.dot(a, b, preferred_element_type=jnp.float32)


def _fism_body(inp_ref, adj_ref,
               eWg0, ebg0, eWc0, ebc0, eWg1, ebg1, eWc1, ebc1,
               dWg0, dbg0, dWc0, dbc0, dWg1, dbg1, dWc1, dbc1,
               wproj, bproj,
               out_ref, h0_ref, h1_ref, src_ref, d1_ref, d2_ref):
    a = adj_ref[...]
    d1 = jnp.maximum(jnp.sum(a, axis=1, keepdims=True), 1e-8)
    sa = (a / d1).T
    at = a.T
    d2 = jnp.maximum(jnp.sum(at, axis=1, keepdims=True), 1e-8)
    sb = (at / d2).T

    def dconv(x0, w_ref, b_ref):
        src_ref[...] = x0
        acc = _mm(x0, w_ref[0])
        mi = 1
        for s in (sa, sb):
            def bstep(b, _):
                sl = pl.ds(b * _NP, _NP)
                x1b = _mm(s, src_ref[sl, :])
                d1_ref[sl, :] = x1b
                d2_ref[sl, :] = 2.0 * _mm(s, x1b) - src_ref[sl, :]
                return 0
            jax.lax.fori_loop(0, _B, bstep, 0, unroll=4)
            acc = acc + _mm(d1_ref[...], w_ref[mi]) + _mm(d2_ref[...], w_ref[mi + 1])
            mi += 2
        return acc + b_ref[...]

    zpad0 = jnp.zeros((_R, _C - 1 - _U), jnp.float32)

    def cell0(x, h, wg, bg, wc, bc):
        cat = jnp.concatenate([x, h, zpad0], axis=1)
        val = jax.nn.sigmoid(dconv(cat, wg, bg))
        r = val[:, :_U]
        u = val[:, _U:]
        cat2 = jnp.concatenate([x, r * h, zpad0], axis=1)
        c = jnp.tanh(dconv(cat2, wc, bc))
        return u * h + (1.0 - u) * c

    def cell1(x, h, wg, bg, wc, bc):
        cat = jnp.concatenate([x, h], axis=1)
        val = jax.nn.sigmoid(dconv(cat, wg, bg))
        r = val[:, :_U]
        u = val[:, _U:]
        cat2 = jnp.concatenate([x, r * h], axis=1)
        c = jnp.tanh(dconv(cat2, wc, bc))
        return u * h + (1.0 - u) * c

    h0_ref[...] = jnp.zeros((_R, _U), jnp.float32)
    h1_ref[...] = jnp.zeros((_R, _U), jnp.float32)

    def enc_body(t, carry):
        x = inp_ref[t]
        nh0 = cell0(x, h0_ref[...], eWg0, ebg0, eWc0, ebc0)
        nh1 = cell1(nh0, h1_ref[...], eWg1, ebg1, eWc1, ebc1)
        h0_ref[...] = nh0
        h1_ref[...] = nh1
        return carry

    jax.lax.fori_loop(0, _SEQ, enc_body, 0, unroll=False)

    def dec_body(t, dec_in):
        nh0 = cell0(dec_in, h0_ref[...], dWg0, dbg0, dWc0, dbc0)
        nh1 = cell1(nh0, h1_ref[...], dWg1, dbg1, dWc1, dbc1)
        h0_ref[...] = nh0
        h1_ref[...] = nh1
        proj = _mm(nh1, wproj[...]) + bproj[...]
        out_ref[t] = proj
        return proj

    jax.lax.fori_loop(0, _HOR, dec_body,
                      jnp.zeros((_R, 1), jnp.float32), unroll=False)


def _reorder_w(w, cin, out):
    w = w.reshape(cin, _M, out).transpose(1, 0, 2)
    return jnp.pad(w, ((0, 0), (0, _C - cin), (0, 0)))


def kernel(inputs, adj, params):
    inp = jnp.pad(inputs, ((0, 0), (0, 0), (0, _NP - _N)))
    inp = inp.reshape(_SEQ, _R, 1)
    adj_p = jnp.pad(adj, ((0, _NP - _N), (0, _NP - _N)))

    args = [inp, adj_p]
    for mdl in ("enc", "dec"):
        for l in range(2):
            cin = (1 if l == 0 else _U) + _U
            args.append(_reorder_w(params[f"{mdl}_Wg{l}"], cin, 2 * _U))
            args.append(params[f"{mdl}_bg{l}"].reshape(1, 2 * _U))
            args.append(_reorder_w(params[f"{mdl}_Wc{l}"], cin, _U))
            args.append(params[f"{mdl}_bc{l}"].reshape(1, _U))
    args.append(params["W_proj"])
    args.append(params["b_proj"].reshape(1, 1))

    out = pl.pallas_call(
        _fism_body,
        out_shape=jax.ShapeDtypeStruct((_HOR, _R, 1), jnp.float32),
        scratch_shapes=[pltpu.VMEM((_R, _U), jnp.float32),
                        pltpu.VMEM((_R, _U), jnp.float32),
                        pltpu.VMEM((_R, _C), jnp.float32),
                        pltpu.VMEM((_R, _C), jnp.float32),
                        pltpu.VMEM((_R, _C), jnp.float32)],
    )(*args)

    return out.reshape(_HOR, _B, _NP)[:, :, :_N]


# speedup vs baseline: 5.7821x; 1.0000x over previous
import jax
import jax.numpy as jnp
from jax.experimental import pallas as pl
from jax.experimental.pallas import tpu as pltpu

_N = 207
_NP = 208
_B = 8
_R = _B * _NP
_U = 64
_C = 128
_SEQ = 12
_HOR = 12
_M = 5


def _mm(a, b):
    return jnp.dot(a.astype(jnp.bfloat16), b.astype(jnp.bfloat16),
                   preferred_element_type=jnp.float32)


def _fism_body(inp_ref, adj_ref,
               eWg0, ebg0, eWc0, ebc0, eWg1, ebg1, eWc1, ebc1,
               dWg0, dbg0, dWc0, dbc0, dWg1, dbg1, dWc1, dbc1,
               wproj, bproj,
               out_ref, h0_ref, h1_ref, src_ref, d1_ref, d2_ref):
    a = adj_ref[...]
    d1 = jnp.maximum(jnp.sum(a, axis=1, keepdims=True), 1e-8)
    sa = (a / d1).T
    at = a.T
    d2 = jnp.maximum(jnp.sum(at, axis=1, keepdims=True), 1e-8)
    sb = (at / d2).T

    def dconv(x0, w_ref, b_ref):
        src_ref[...] = x0
        acc = _mm(x0, w_ref[0])
        mi = 1
        for s in (sa, sb):
            def bstep(b, _):
                sl = pl.ds(b * _NP, _NP)
                x1b = _mm(s, src_ref[sl, :])
                d1_ref[sl, :] = x1b
                d2_ref[sl, :] = 2.0 * _mm(s, x1b) - src_ref[sl, :]
                return 0
            jax.lax.fori_loop(0, _B, bstep, 0, unroll=4)
            acc = acc + _mm(d1_ref[...], w_ref[mi]) + _mm(d2_ref[...], w_ref[mi + 1])
            mi += 2
        return acc + b_ref[...]

    zpad0 = jnp.zeros((_R, _C - 1 - _U), jnp.float32)

    def cell0(x, h, wg, bg, wc, bc):
        cat = jnp.concatenate([x, h, zpad0], axis=1)
        val = jax.nn.sigmoid(dconv(cat, wg, bg))
        r = val[:, :_U]
        u = val[:, _U:]
        cat2 = jnp.concatenate([x, r * h, zpad0], axis=1)
        c = jnp.tanh(dconv(cat2, wc, bc))
        return u * h + (1.0 - u) * c

    def cell1(x, h, wg, bg, wc, bc):
        cat = jnp.concatenate([x, h], axis=1)
        val = jax.nn.sigmoid(dconv(cat, wg, bg))
        r = val[:, :_U]
        u = val[:, _U:]
        cat2 = jnp.concatenate([x, r * h], axis=1)
        c = jnp.tanh(dconv(cat2, wc, bc))
        return u * h + (1.0 - u) * c

    h0_ref[...] = jnp.zeros((_R, _U), jnp.float32)
    h1_ref[...] = jnp.zeros((_R, _U), jnp.float32)

    def enc_body(t, carry):
        x = inp_ref[t]
        nh0 = cell0(x, h0_ref[...], eWg0, ebg0, eWc0, ebc0)
        nh1 = cell1(nh0, h1_ref[...], eWg1, ebg1, eWc1, ebc1)
        h0_ref[...] = nh0
        h1_ref[...] = nh1
        return carry

    jax.lax.fori_loop(0, _SEQ, enc_body, 0, unroll=False)

    def dec_body(t, dec_in):
        nh0 = cell0(dec_in, h0_ref[...], dWg0, dbg0, dWc0, dbc0)
        nh1 = cell1(nh0, h1_ref[...], dWg1, dbg1, dWc1, dbc1)
        h0_ref[...] = nh0
        h1_ref[...] = nh1
        proj = _mm(nh1, wproj[...]) + bproj[...]
        out_ref[t] = proj
        return proj

    jax.lax.fori_loop(0, _HOR, dec_body,
                      jnp.zeros((_R, 1), jnp.float32), unroll=False)


def _reorder_w(w, cin, out):
    w = w.reshape(cin, _M, out).transpose(1, 0, 2)
    return jnp.pad(w, ((0, 0), (0, _C - cin), (0, 0)))


def kernel(inputs, adj, params):
    inp = jnp.pad(inputs, ((0, 0), (0, 0), (0, _NP - _N)))
    inp = inp.reshape(_SEQ, _R, 1)
    adj_p = jnp.pad(adj, ((0, _NP - _N), (0, _NP - _N)))

    args = [inp, adj_p]
    for mdl in ("enc", "dec"):
        for l in range(2):
            cin = (1 if l == 0 else _U) + _U
            args.append(_reorder_w(params[f"{mdl}_Wg{l}"], cin, 2 * _U))
            args.append(params[f"{mdl}_bg{l}"].reshape(1, 2 * _U))
            args.append(_reorder_w(params[f"{mdl}_Wc{l}"], cin, _U))
            args.append(params[f"{mdl}_bc{l}"].reshape(1, _U))
    args.append(params["W_proj"])
    args.append(params["b_proj"].reshape(1, 1))

    out = pl.pallas_call(
        _fism_body,
        out_shape=jax.ShapeDtypeStruct((_HOR, _R, 1), jnp.float32),
        scratch_shapes=[pltpu.VMEM((_R, _U), jnp.float32),
                        pltpu.VMEM((_R, _U), jnp.float32),
                        pltpu.VMEM((_R, _C), jnp.float32),
                        pltpu.VMEM((_R, _C), jnp.float32),
                        pltpu.VMEM((_R, _C), jnp.float32)],
    )(*args)

    return out.reshape(_HOR, _B, _NP)[:, :, :_N]
